# Initial kernel scaffold; baseline (speedup 1.0000x reference)
import jax
import jax.numpy as jnp
from jax.experimental import pallas as pl


def kernel(x, x1, edge_weight, W_rel, b_rel, W_root, W1, b1, W4, b4, W2, b2, W3, b3, edge_index, loc_num):
    raise NotImplementedError("write your pallas kernel here")


# speedup vs baseline: 3.2591x; 3.2591x over previous
import functools
import jax
import jax.numpy as jnp
from jax import lax
from jax.experimental import pallas as pl
from jax.experimental.pallas import tpu as pltpu
from jax.experimental.pallas import tpu_sc as plsc

D_H = 128
CHUNK = 128
NC, NS = 2, 16
NW = NC * NS
LANES = 16


def _make_sc_segment_sum(n_nodes, e_pad):
  chunks_per_tile = e_pad // (NW * CHUNK)
  rows_per_tile = ((n_nodes + NS - 1) // NS + 7) // 8 * 8
  n_pad = rows_per_tile * NS
  full_zero_chunks = rows_per_tile // CHUNK
  rem_zero = rows_per_tile % CHUNK
  mesh = plsc.VectorSubcoreMesh(core_axis_name="c", subcore_axis_name="s",
                                num_cores=NC, num_subcores=NS)

  @functools.partial(
      pl.kernel,
      out_type=[jax.ShapeDtypeStruct((n_pad, D_H), jnp.float32)] * NC,
      mesh=mesh,
      scratch_types=[
          pltpu.VMEM((CHUNK,), jnp.int32),
          pltpu.VMEM((CHUNK,), jnp.int32),
          pltpu.VMEM((CHUNK,), jnp.float32),
          pltpu.VMEM((CHUNK, D_H), jnp.float32),
          pltpu.VMEM_SHARED((n_pad, D_H), jnp.float32),
          pltpu.SemaphoreType.DMA,
      ],
  )
  def sc_kernel(y_hbm, src_hbm, dst_hbm, ew_hbm, out0_hbm, out1_hbm,
                src_v, dst_v, ew_v, rows_v, agg_sh, sem):
    c = lax.axis_index("c")
    s = lax.axis_index("s")
    wid = c * NS + s
    row0 = s * rows_per_tile

    zero16 = jnp.zeros((LANES,), jnp.float32)

    def zrow(i, carry):
      for f in range(D_H // LANES):
        rows_v[i, pl.ds(f * LANES, LANES)] = zero16
      return carry

    lax.fori_loop(0, CHUNK, zrow, 0)
    for k in range(full_zero_chunks):
      pltpu.sync_copy(rows_v, agg_sh.at[pl.ds(row0 + k * CHUNK, CHUNK)])
    if rem_zero:
      pltpu.sync_copy(rows_v.at[pl.ds(0, rem_zero)],
                      agg_sh.at[pl.ds(row0 + full_zero_chunks * CHUNK, rem_zero)])
    plsc.subcore_barrier()

    ebase = wid * chunks_per_tile * CHUNK

    def chunk_body(ci, carry):
      base = ebase + ci * CHUNK
      pltpu.sync_copy(src_hbm.at[pl.ds(base, CHUNK)], src_v)
      pltpu.sync_copy(dst_hbm.at[pl.ds(base, CHUNK)], dst_v)
      pltpu.sync_copy(ew_hbm.at[pl.ds(base, CHUNK)], ew_v)
      pltpu.async_copy(y_hbm.at[src_v], rows_v, sem).wait()

      def scale(g, inner):
        wvec = ew_v[pl.ds(g * LANES, LANES)]
        for j in range(LANES):
          wv = jnp.full((LANES,), wvec[j], jnp.float32)
          e = g * LANES + j
          for f in range(D_H // LANES):
            sl = pl.ds(f * LANES, LANES)
            rows_v[e, sl] = rows_v[e, sl] * wv
        return inner

      lax.fori_loop(0, CHUNK // LANES, scale, 0)
      pltpu.sync_copy(rows_v, agg_sh.at[dst_v], add=True)
      return carry

    lax.fori_loop(0, chunks_per_tile, chunk_body, 0)
    plsc.subcore_barrier()

    @pl.when(c == 0)
    def _():
      pltpu.sync_copy(agg_sh.at[pl.ds(row0, rows_per_tile)],
                      out0_hbm.at[pl.ds(row0, rows_per_tile)])

    @pl.when(c == 1)
    def _():
      pltpu.sync_copy(agg_sh.at[pl.ds(row0, rows_per_tile)],
                      out1_hbm.at[pl.ds(row0, rows_per_tile)])

  return sc_kernel


def _mm_body(x_ref, w_ref, y_ref, r_ref):
  acc = jnp.dot(x_ref[...], w_ref[...], preferred_element_type=jnp.float32)
  y_ref[...] = acc[:, :D_H]
  r_ref[...] = acc[:, D_H:]


def _head_body(agg0, agg1, r, x1, brel, w1t, b1, w4t, b4, w2et, w2x, b2, w3t, b3,
               out_ref, emb_ref):
  h = jnp.maximum(agg0[...] + agg1[...] + r[...] + brel[...], 0.0)
  h1 = jnp.maximum(
      jnp.dot(h, w1t[...], preferred_element_type=jnp.float32) + b1[...], 0.0)
  emb = jax.nn.sigmoid(
      jnp.dot(h1, w4t[...], preferred_element_type=jnp.float32) + b4[...])
  emb_ref[...] = emb
  h2 = jax.nn.sigmoid(
      jnp.dot(emb, w2et[...], preferred_element_type=jnp.float32)
      + x1[...] * w2x[...] + b2[...])
  out_ref[...] = jnp.maximum(
      jnp.dot(h2, w3t[...], preferred_element_type=jnp.float32) + b3[...], 0.0)


def kernel(x, x1, edge_weight, W_rel, b_rel, W_root, W1, b1, W4, b4, W2, b2, W3,
           b3, edge_index, loc_num):
  n, d = x.shape
  e = edge_index.shape[1]
  blk = 1000
  nblk = n // blk

  wcat = jnp.concatenate([W_rel, W_root], axis=0).T
  y, r = pl.pallas_call(
      _mm_body,
      grid=(nblk,),
      in_specs=[
          pl.BlockSpec((blk, d), lambda i: (i, 0)),
          pl.BlockSpec((d, 2 * D_H), lambda i: (0, 0)),
      ],
      out_specs=[
          pl.BlockSpec((blk, D_H), lambda i: (i, 0)),
          pl.BlockSpec((blk, D_H), lambda i: (i, 0)),
      ],
      out_shape=[
          jax.ShapeDtypeStruct((n, D_H), jnp.float32),
          jax.ShapeDtypeStruct((n, D_H), jnp.float32),
      ],
  )(x, wcat)

  e_pad = ((e + NW * CHUNK - 1) // (NW * CHUNK)) * (NW * CHUNK)
  pad = e_pad - e
  src = edge_index[0]
  dst = edge_index[1]
  ew = edge_weight
  if pad:
    zi = jnp.zeros((pad,), jnp.int32)
    src = jnp.concatenate([src, zi])
    dst = jnp.concatenate([dst, zi])
    ew = jnp.concatenate([ew, jnp.zeros((pad,), jnp.float32)])
  agg0, agg1 = _make_sc_segment_sum(n, e_pad)(y, src, dst, ew)

  delta = (jnp.asarray(loc_num) - n).astype(jnp.float32)
  b_eff = (b_rel + delta).reshape(1, D_H)
  x1c = x1.reshape(n, 1)
  w1t = W1.T
  b1r = b1.reshape(1, 32)
  w4t = W4.T
  b4r = b4.reshape(1, 8)
  w2et = W2[:, :8].T
  w2x = W2[:, 8].reshape(1, 4)
  b2r = b2.reshape(1, 4)
  w3t = W3.T
  b3r = b3.reshape(1, 1)

  out, emb = pl.pallas_call(
      _head_body,
      grid=(nblk,),
      in_specs=[
          pl.BlockSpec((blk, D_H), lambda i: (i, 0)),
          pl.BlockSpec((blk, D_H), lambda i: (i, 0)),
          pl.BlockSpec((blk, D_H), lambda i: (i, 0)),
          pl.BlockSpec((blk, 1), lambda i: (i, 0)),
          pl.BlockSpec((1, D_H), lambda i: (0, 0)),
          pl.BlockSpec((D_H, 32), lambda i: (0, 0)),
          pl.BlockSpec((1, 32), lambda i: (0, 0)),
          pl.BlockSpec((32, 8), lambda i: (0, 0)),
          pl.BlockSpec((1, 8), lambda i: (0, 0)),
          pl.BlockSpec((8, 4), lambda i: (0, 0)),
          pl.BlockSpec((1, 4), lambda i: (0, 0)),
          pl.BlockSpec((1, 4), lambda i: (0, 0)),
          pl.BlockSpec((4, 1), lambda i: (0, 0)),
          pl.BlockSpec((1, 1), lambda i: (0, 0)),
      ],
      out_specs=[
          pl.BlockSpec((blk, 1), lambda i: (i, 0)),
          pl.BlockSpec((blk, 8), lambda i: (i, 0)),
      ],
      out_shape=[
          jax.ShapeDtypeStruct((n, 1), jnp.float32),
          jax.ShapeDtypeStruct((n, 8), jnp.float32),
      ],
  )(agg0, agg1, r, x1c, b_eff, w1t, b1r, w4t, b4r, w2et, w2x, b2r, w3t, b3r)
  return (out, emb)


# speedup vs baseline: 4.1022x; 1.2587x over previous
import functools
import jax
import jax.numpy as jnp
from jax import lax
from jax.experimental import pallas as pl
from jax.experimental.pallas import tpu as pltpu
from jax.experimental.pallas import tpu_sc as plsc

D_H = 128
CHUNK = 128
NC, NS = 2, 16
NW = NC * NS
LANES = 16


def _make_sc_segment_sum(n_nodes, e_pad):
  chunks_per_tile = e_pad // (NW * CHUNK)
  rows_per_tile = ((n_nodes + NS - 1) // NS + 7) // 8 * 8
  n_pad = rows_per_tile * NS
  full_zero_chunks = rows_per_tile // CHUNK
  rem_zero = rows_per_tile % CHUNK
  mesh = plsc.VectorSubcoreMesh(core_axis_name="c", subcore_axis_name="s",
                                num_cores=NC, num_subcores=NS)

  edges_per_tile = chunks_per_tile * CHUNK
  assert chunks_per_tile % 2 == 0
  npairs = chunks_per_tile // 2

  @functools.partial(
      pl.kernel,
      out_type=[jax.ShapeDtypeStruct((n_pad, D_H), jnp.float32)] * NC,
      mesh=mesh,
      scratch_types=[
          pltpu.VMEM((edges_per_tile,), jnp.int32),
          pltpu.VMEM((chunks_per_tile, 1, CHUNK), jnp.int32),
          pltpu.VMEM((edges_per_tile,), jnp.float32),
          pltpu.VMEM((CHUNK, D_H), jnp.float32),
          pltpu.VMEM((CHUNK, D_H), jnp.float32),
          pltpu.VMEM_SHARED((n_pad, D_H), jnp.float32),
          pltpu.SemaphoreType.DMA,
          pltpu.SemaphoreType.DMA,
          pltpu.SemaphoreType.DMA,
          pltpu.SemaphoreType.DMA,
          pltpu.SemaphoreType.DMA,
      ],
  )
  def sc_kernel(y_hbm, src_hbm, dst_hbm, ew_hbm, out0_hbm, out1_hbm,
                src_v, dst_v, ew_v, rows_a, rows_b, agg_sh,
                gs_a, gs_b, ss_a, ss_b, is_sem):
    c = lax.axis_index("c")
    s = lax.axis_index("s")
    wid = c * NS + s
    row0 = s * rows_per_tile
    ebase = wid * edges_per_tile

    ia = pltpu.async_copy(src_hbm.at[pl.ds(ebase, edges_per_tile)], src_v, is_sem)
    ib = pltpu.async_copy(
        dst_hbm.at[pl.ds(wid * chunks_per_tile, chunks_per_tile)], dst_v, is_sem)
    ic = pltpu.async_copy(ew_hbm.at[pl.ds(ebase, edges_per_tile)], ew_v, is_sem)

    zero16 = jnp.zeros((LANES,), jnp.float32)

    def zrow(i, carry):
      for f in range(D_H // LANES):
        rows_a[i, pl.ds(f * LANES, LANES)] = zero16
      return carry

    lax.fori_loop(0, CHUNK, zrow, 0)
    for k in range(full_zero_chunks):
      pltpu.sync_copy(rows_a, agg_sh.at[pl.ds(row0 + k * CHUNK, CHUNK)])
    if rem_zero:
      pltpu.sync_copy(rows_a.at[pl.ds(0, rem_zero)],
                      agg_sh.at[pl.ds(row0 + full_zero_chunks * CHUNK, rem_zero)])
    ia.wait()
    ib.wait()
    ic.wait()
    plsc.subcore_barrier()

    def issue_gather(ci, buf, gsem):
      return pltpu.async_copy(
          y_hbm.at[src_v.at[pl.ds(ci * CHUNK, CHUNK)]], buf, gsem)

    def issue_scatter(ci, buf, ssem):
      return pltpu.async_copy(buf, agg_sh.at[dst_v.at[ci, 0]], ssem, add=True)

    def do_scale(ci, buf):
      def scale(g, inner):
        wvec = ew_v[pl.ds(ci * CHUNK + g * LANES, LANES)]
        for j in range(LANES):
          wv = jnp.full((LANES,), wvec[j], jnp.float32)
          e = g * LANES + j
          for f in range(D_H // LANES):
            sl = pl.ds(f * LANES, LANES)
            buf[e, sl] = buf[e, sl] * wv
        return inner

      lax.fori_loop(0, CHUNK // LANES, scale, 0)

    issue_gather(0, rows_a, gs_a)
    issue_gather(1, rows_b, gs_b)

    def pair_body(p, carry):
      ci = 2 * p
      pltpu.make_async_copy(y_hbm.at[src_v.at[pl.ds(0, CHUNK)]], rows_a,
                            gs_a).wait()
      do_scale(ci, rows_a)
      issue_scatter(ci, rows_a, ss_a)
      pltpu.make_async_copy(y_hbm.at[src_v.at[pl.ds(0, CHUNK)]], rows_b,
                            gs_b).wait()
      do_scale(ci + 1, rows_b)
      issue_scatter(ci + 1, rows_b, ss_b)

      @pl.when(p < npairs - 1)
      def _():
        pltpu.make_async_copy(rows_a, agg_sh.at[dst_v.at[0, 0]], ss_a).wait()
        issue_gather(ci + 2, rows_a, gs_a)
        pltpu.make_async_copy(rows_b, agg_sh.at[dst_v.at[0, 0]], ss_b).wait()
        issue_gather(ci + 3, rows_b, gs_b)

      return carry

    lax.fori_loop(0, npairs, pair_body, 0)
    pltpu.make_async_copy(rows_a, agg_sh.at[dst_v.at[0, 0]], ss_a).wait()
    pltpu.make_async_copy(rows_b, agg_sh.at[dst_v.at[0, 0]], ss_b).wait()
    plsc.subcore_barrier()

    @pl.when(c == 0)
    def _():
      pltpu.sync_copy(agg_sh.at[pl.ds(row0, rows_per_tile)],
                      out0_hbm.at[pl.ds(row0, rows_per_tile)])

    @pl.when(c == 1)
    def _():
      pltpu.sync_copy(agg_sh.at[pl.ds(row0, rows_per_tile)],
                      out1_hbm.at[pl.ds(row0, rows_per_tile)])

  return sc_kernel


def _mm_body(x_ref, w_ref, y_ref, r_ref):
  acc = jnp.dot(x_ref[...], w_ref[...], preferred_element_type=jnp.float32)
  y_ref[...] = acc[:, :D_H]
  r_ref[...] = acc[:, D_H:]


def _head_body(agg0, agg1, r, x1, brel, w1t, b1, w4t, b4, w2et, w2x, b2, w3t, b3,
               out_ref, emb_ref):
  h = jnp.maximum(agg0[...] + agg1[...] + r[...] + brel[...], 0.0)
  h1 = jnp.maximum(
      jnp.dot(h, w1t[...], preferred_element_type=jnp.float32) + b1[...], 0.0)
  emb = jax.nn.sigmoid(
      jnp.dot(h1, w4t[...], preferred_element_type=jnp.float32) + b4[...])
  emb_ref[...] = emb
  h2 = jax.nn.sigmoid(
      jnp.dot(emb, w2et[...], preferred_element_type=jnp.float32)
      + x1[...] * w2x[...] + b2[...])
  out_ref[...] = jnp.maximum(
      jnp.dot(h2, w3t[...], preferred_element_type=jnp.float32) + b3[...], 0.0)


def kernel(x, x1, edge_weight, W_rel, b_rel, W_root, W1, b1, W4, b4, W2, b2, W3,
           b3, edge_index, loc_num):
  n, d = x.shape
  e = edge_index.shape[1]
  blk = 1000
  nblk = n // blk

  wcat = jnp.concatenate([W_rel, W_root], axis=0).T
  y, r = pl.pallas_call(
      _mm_body,
      grid=(nblk,),
      in_specs=[
          pl.BlockSpec((blk, d), lambda i: (i, 0)),
          pl.BlockSpec((d, 2 * D_H), lambda i: (0, 0)),
      ],
      out_specs=[
          pl.BlockSpec((blk, D_H), lambda i: (i, 0)),
          pl.BlockSpec((blk, D_H), lambda i: (i, 0)),
      ],
      out_shape=[
          jax.ShapeDtypeStruct((n, D_H), jnp.float32),
          jax.ShapeDtypeStruct((n, D_H), jnp.float32),
      ],
  )(x, wcat)

  e_pad = ((e + NW * CHUNK - 1) // (NW * CHUNK)) * (NW * CHUNK)
  pad = e_pad - e
  src = edge_index[0]
  dst = edge_index[1]
  ew = edge_weight
  if pad:
    zi = jnp.zeros((pad,), jnp.int32)
    src = jnp.concatenate([src, zi])
    dst = jnp.concatenate([dst, zi])
    ew = jnp.concatenate([ew, jnp.zeros((pad,), jnp.float32)])
  dst3 = dst.reshape(e_pad // CHUNK, 1, CHUNK)
  agg0, agg1 = _make_sc_segment_sum(n, e_pad)(y, src, dst3, ew)

  delta = (jnp.asarray(loc_num) - n).astype(jnp.float32)
  b_eff = (b_rel + delta).reshape(1, D_H)
  x1c = x1.reshape(n, 1)
  w1t = W1.T
  b1r = b1.reshape(1, 32)
  w4t = W4.T
  b4r = b4.reshape(1, 8)
  w2et = W2[:, :8].T
  w2x = W2[:, 8].reshape(1, 4)
  b2r = b2.reshape(1, 4)
  w3t = W3.T
  b3r = b3.reshape(1, 1)

  out, emb = pl.pallas_call(
      _head_body,
      grid=(nblk,),
      in_specs=[
          pl.BlockSpec((blk, D_H), lambda i: (i, 0)),
          pl.BlockSpec((blk, D_H), lambda i: (i, 0)),
          pl.BlockSpec((blk, D_H), lambda i: (i, 0)),
          pl.BlockSpec((blk, 1), lambda i: (i, 0)),
          pl.BlockSpec((1, D_H), lambda i: (0, 0)),
          pl.BlockSpec((D_H, 32), lambda i: (0, 0)),
          pl.BlockSpec((1, 32), lambda i: (0, 0)),
          pl.BlockSpec((32, 8), lambda i: (0, 0)),
          pl.BlockSpec((1, 8), lambda i: (0, 0)),
          pl.BlockSpec((8, 4), lambda i: (0, 0)),
          pl.BlockSpec((1, 4), lambda i: (0, 0)),
          pl.BlockSpec((1, 4), lambda i: (0, 0)),
          pl.BlockSpec((4, 1), lambda i: (0, 0)),
          pl.BlockSpec((1, 1), lambda i: (0, 0)),
      ],
      out_specs=[
          pl.BlockSpec((blk, 1), lambda i: (i, 0)),
          pl.BlockSpec((blk, 8), lambda i: (i, 0)),
      ],
      out_shape=[
          jax.ShapeDtypeStruct((n, 1), jnp.float32),
          jax.ShapeDtypeStruct((n, 8), jnp.float32),
      ],
  )(agg0, agg1, r, x1c, b_eff, w1t, b1r, w4t, b4r, w2et, w2x, b2r, w3t, b3r)
  return (out, emb)
